# baseline (speedup 1.0000x reference)
import functools

import jax
import jax.numpy as jnp
from jax import lax
from jax.experimental import pallas as pl
from jax.experimental.pallas import tpu as pltpu
from jax.experimental.pallas import tpu_sc as plsc

NUM_EMB = 1_000_000
D = 64
B = 16384 * 20
NC, NS = 2, 16
NW = NC * NS
BPW = B // NW
C = 128
NCHUNK = BPW // C

_mesh = plsc.VectorSubcoreMesh(core_axis_name="c", subcore_axis_name="s")


@functools.partial(
    pl.kernel,
    out_type=jax.ShapeDtypeStruct((B, D), jnp.float32),
    mesh=_mesh,
    scratch_types=[
        pltpu.VMEM((NCHUNK, C), jnp.int32),
        pltpu.VMEM((C, D), jnp.float32),
        pltpu.SemaphoreType.DMA,
    ],
    compiler_params=pltpu.CompilerParams(use_tc_tiling_on_sc=False),
)
def _emb_lookup(idx_hbm, table_hbm, out_hbm, idx_v, rows_v, gsem):
    wid = lax.axis_index("s") * NC + lax.axis_index("c")
    base = wid * BPW
    pltpu.sync_copy(idx_hbm.at[wid], idx_v)

    def body(j, carry):
        pltpu.async_copy(table_hbm.at[idx_v.at[j]], rows_v, gsem).wait()
        pltpu.sync_copy(rows_v, out_hbm.at[pl.ds(base + j * C, C)])
        return carry

    lax.fori_loop(0, NCHUNK, body, 0)


def kernel(token_ids, embeddings):
    idx = token_ids.reshape(NW, NCHUNK, C).astype(jnp.int32)
    out = _emb_lookup(idx, embeddings)
    return out.reshape(*token_ids.shape, D)


# speedup vs baseline: 1.0638x; 1.0638x over previous
import functools

import jax
import jax.numpy as jnp
from jax import lax
from jax.experimental import pallas as pl
from jax.experimental.pallas import tpu as pltpu
from jax.experimental.pallas import tpu_sc as plsc

NUM_EMB = 1_000_000
D = 64
B = 16384 * 20
NC, NS = 2, 16
NW = NC * NS
BPW = B // NW
C = 128
NCHUNK = BPW // C
NBUF = 8
GA = 4
KS = NBUF - GA
NGROUP = NCHUNK // NBUF

_mesh = plsc.VectorSubcoreMesh(core_axis_name="c", subcore_axis_name="s")


@functools.partial(
    pl.kernel,
    out_type=jax.ShapeDtypeStruct((B, D), jnp.float32),
    mesh=_mesh,
    scratch_types=[
        pltpu.VMEM((NCHUNK, C), jnp.int32),
        pltpu.VMEM((NBUF, C, D), jnp.float32),
        pltpu.SemaphoreType.DMA,
        pltpu.SemaphoreType.DMA,
    ],
    compiler_params=pltpu.CompilerParams(use_tc_tiling_on_sc=False),
)
def _emb_lookup(idx_hbm, table_hbm, out_hbm, idx_v, rows_v, gsem, ssem):
    wid = lax.axis_index("s") * NC + lax.axis_index("c")
    base = wid * BPW
    pltpu.sync_copy(idx_hbm.at[wid], idx_v)

    def gather(chunk, slot):
        pltpu.async_copy(table_hbm.at[idx_v.at[chunk]], rows_v.at[slot], gsem)

    def wait_gather(slot):
        pltpu.make_async_copy(
            table_hbm.at[idx_v.at[0]], rows_v.at[slot], gsem).wait()

    def store(chunk, slot):
        pltpu.async_copy(
            rows_v.at[slot], out_hbm.at[pl.ds(base + chunk * C, C)], ssem)

    def wait_store(slot):
        pltpu.make_async_copy(
            rows_v.at[slot], out_hbm.at[pl.ds(base, C)], ssem).wait()

    for b in range(GA):
        gather(b, b)

    for b in range(NBUF):
        wait_gather(b)
        store(b, b)
        if b >= KS:
            wait_store((b - KS) % NBUF)
        gather(b + GA, (b + GA) % NBUF)

    def body(g, carry):
        for b in range(NBUF):
            j = g * NBUF + b
            slot = b
            wait_gather(slot)
            store(j, slot)
            wait_store((b - KS) % NBUF)
            gather(jnp.minimum(j + GA, NCHUNK - 1), (b + GA) % NBUF)
        return carry

    lax.fori_loop(1, NGROUP, body, 0)

    for b in range(GA):
        wait_gather((NCHUNK + b) % NBUF)
    for b in range(KS):
        wait_store((NCHUNK - KS + b) % NBUF)


def kernel(token_ids, embeddings):
    idx = token_ids.reshape(NW, NCHUNK, C).astype(jnp.int32)
    out = _emb_lookup(idx, embeddings)
    return out.reshape(*token_ids.shape, D)
